# baseline (speedup 1.0000x reference)
import functools

import jax
import jax.numpy as jnp
from jax import lax
from jax.experimental import pallas as pl
from jax.experimental.pallas import tpu as pltpu
from jax.experimental.pallas import tpu_sc as plsc

D_ENT = 128
D_REL = 128
NC = 2
NS = 16
GB = 8
CH = 128


def _sc_gather_entities(ent, h_idx, t_idx):
    B = h_idx.shape[0]
    nw = NC * NS
    bpw = B // nw
    assert B % (8 * nw) == 0
    mesh = plsc.VectorSubcoreMesh(core_axis_name="c", subcore_axis_name="s")

    @functools.partial(
        pl.kernel,
        out_type=(
            jax.ShapeDtypeStruct((B, D_ENT), jnp.float32),
            jax.ShapeDtypeStruct((B, D_ENT), jnp.float32),
        ),
        mesh=mesh,
        scratch_types=[
            pltpu.VMEM((bpw,), jnp.int32),
            pltpu.VMEM((bpw,), jnp.int32),
            pltpu.VMEM((bpw, D_ENT), jnp.float32),
            pltpu.VMEM((bpw, D_ENT), jnp.float32),
            pltpu.SemaphoreType.DMA,
            pltpu.SemaphoreType.DMA,
        ],
    )
    def k(ent_hbm, h_hbm, t_hbm, hout, tout, hi_v, ti_v, hr_v, tr_v, s1, s2):
        wid = lax.axis_index("s") * NC + lax.axis_index("c")
        base = wid * bpw
        pltpu.sync_copy(h_hbm.at[pl.ds(base, bpw)], hi_v)
        pltpu.sync_copy(t_hbm.at[pl.ds(base, bpw)], ti_v)
        c1 = pltpu.async_copy(ent_hbm.at[hi_v], hr_v, s1)
        c2 = pltpu.async_copy(ent_hbm.at[ti_v], tr_v, s2)
        c1.wait()
        c2.wait()
        pltpu.sync_copy(hr_v, hout.at[pl.ds(base, bpw)])
        pltpu.sync_copy(tr_v, tout.at[pl.ds(base, bpw)])

    return k(ent, h_idx, t_idx)


def _tc_score_body(starts_ref, t_ref, rel_ref, h_ref, tr_ref, out_ref,
                   d_ref, acc_ref):
    B = h_ref.shape[0]
    k = pl.program_id(0)
    nsteps = pl.num_programs(0)

    @pl.when(k == 0)
    def _():
        for c in range(B // CH):
            sl = pl.ds(c * CH, CH)
            d_ref[sl, :] = (h_ref[sl, :] - tr_ref[sl, :]).astype(jnp.bfloat16)
            acc_ref[sl, :] = jnp.zeros((CH, D_REL), jnp.float32)

    lo = starts_ref[k * GB]
    hi = starts_ref[k * GB + GB]

    def chunk(c, _):
        row0 = pl.multiple_of(c * CH, CH)
        d = d_ref[pl.ds(row0, CH), :]
        gl = row0 + lax.broadcasted_iota(jnp.int32, (CH, 1), 0)
        acc = acc_ref[pl.ds(row0, CH), :]
        for g in range(GB):
            s = starts_ref[k * GB + g]
            e = starts_ref[k * GB + g + 1]
            y = lax.dot_general(
                d, t_ref[g], (((1,), (1,)), ((), ())),
                preferred_element_type=jnp.float32,
            )
            a = jnp.abs(y + rel_ref[g, :][None, :])
            m = (gl >= s) & (gl < e)
            acc = acc + jnp.where(m, a, 0.0)
        acc_ref[pl.ds(row0, CH), :] = acc
        return 0

    lax.fori_loop(lo // CH, (hi + CH - 1) // CH, chunk, 0)

    @pl.when(k == nsteps - 1)
    def _():
        for c in range(B // CH):
            sl = pl.ds(c * CH, CH)
            out_ref[sl, :] = jnp.sum(acc_ref[sl, :], axis=1, keepdims=True)


def _tc_score(t3, rel, hrows, trows, starts):
    B = hrows.shape[0]
    nrel = rel.shape[0]
    assert nrel % GB == 0
    grid_spec = pltpu.PrefetchScalarGridSpec(
        num_scalar_prefetch=1,
        grid=(nrel // GB,),
        in_specs=[
            pl.BlockSpec((GB, D_REL, D_ENT), lambda k, st: (k, 0, 0)),
            pl.BlockSpec((GB, D_REL), lambda k, st: (k, 0)),
            pl.BlockSpec((B, D_ENT), lambda k, st: (0, 0)),
            pl.BlockSpec((B, D_ENT), lambda k, st: (0, 0)),
        ],
        out_specs=pl.BlockSpec((B, 1), lambda k, st: (0, 0)),
        scratch_shapes=[
            pltpu.VMEM((B, D_ENT), jnp.bfloat16),
            pltpu.VMEM((B, D_REL), jnp.float32),
        ],
    )
    return pl.pallas_call(
        _tc_score_body,
        grid_spec=grid_spec,
        out_shape=jax.ShapeDtypeStruct((B, 1), jnp.float32),
    )(starts, t3, rel, hrows, trows)


def kernel(predict_h, predict_t, predict_r, ent_embeddings, rel_embeddings,
           transfer_matrix):
    B = predict_h.shape[0]
    nrel = rel_embeddings.shape[0]
    iota = jnp.arange(B, dtype=jnp.int32)
    key = jnp.sort(predict_r * B + iota)
    perm = key % B
    sorted_r = key // B
    del sorted_r
    h_s = jnp.take(predict_h, perm)
    t_s = jnp.take(predict_t, perm)
    rr = jnp.arange(nrel + 1, dtype=jnp.int32)
    starts = jnp.sum(
        (predict_r[None, :] < rr[:, None]).astype(jnp.int32), axis=1
    )
    hrows, trows = _sc_gather_entities(ent_embeddings, h_s, t_s)
    t3 = transfer_matrix.astype(jnp.bfloat16).reshape(nrel, D_REL, D_ENT)
    score_sorted = _tc_score(t3, rel_embeddings, hrows, trows, starts)
    return jnp.zeros((B,), jnp.float32).at[perm].set(score_sorted[:, 0])[:, None]


# speedup vs baseline: 1.1338x; 1.1338x over previous
import functools

import jax
import jax.numpy as jnp
from jax import lax
from jax.experimental import pallas as pl
from jax.experimental.pallas import tpu as pltpu
from jax.experimental.pallas import tpu_sc as plsc

D_ENT = 128
D_REL = 128
NC = 2
NS = 16
GB = 40
SB = 8
CH = 128
SPLIT = 480


def _round_bf16(x):
    xi = lax.bitcast_convert_type(x, jnp.uint32)
    xr = lax.bitcast_convert_type(xi + jnp.uint32(0x8000), jnp.float32)
    return xr.astype(jnp.bfloat16)


def _sc_gather_entities(ent, h_idx, t_idx):
    B = h_idx.shape[0]
    nw = NC * NS
    bpw = B // nw
    assert B % (8 * nw) == 0
    mesh = plsc.VectorSubcoreMesh(core_axis_name="c", subcore_axis_name="s")

    @functools.partial(
        pl.kernel,
        out_type=(
            jax.ShapeDtypeStruct((B, D_ENT), jnp.float32),
            jax.ShapeDtypeStruct((B, D_ENT), jnp.float32),
        ),
        mesh=mesh,
        scratch_types=[
            pltpu.VMEM((bpw,), jnp.int32),
            pltpu.VMEM((bpw,), jnp.int32),
            pltpu.VMEM((bpw, D_ENT), jnp.float32),
            pltpu.VMEM((bpw, D_ENT), jnp.float32),
            pltpu.SemaphoreType.DMA,
            pltpu.SemaphoreType.DMA,
        ],
    )
    def k(ent_hbm, h_hbm, t_hbm, hout, tout, hi_v, ti_v, hr_v, tr_v, s1, s2):
        wid = lax.axis_index("s") * NC + lax.axis_index("c")
        base = wid * bpw
        pltpu.sync_copy(h_hbm.at[pl.ds(base, bpw)], hi_v)
        pltpu.sync_copy(t_hbm.at[pl.ds(base, bpw)], ti_v)
        c1 = pltpu.async_copy(ent_hbm.at[hi_v], hr_v, s1)
        c2 = pltpu.async_copy(ent_hbm.at[ti_v], tr_v, s2)
        c1.wait()
        c2.wait()
        pltpu.sync_copy(hr_v, hout.at[pl.ds(base, bpw)])
        pltpu.sync_copy(tr_v, tout.at[pl.ds(base, bpw)])

    return k(ent, h_idx, t_idx)


def _score_blocks(starts_ref, t_ref, rel_ref, d_ref, acc_ref, k):
    for sb in range(GB // SB):
        lo = starts_ref[k * GB + sb * SB]
        hi = starts_ref[k * GB + sb * SB + SB]

        def chunk(c, _, sb=sb):
            row0 = pl.multiple_of(c * CH, CH)
            d = d_ref[pl.ds(row0, CH), :]
            gl = row0 + lax.broadcasted_iota(jnp.int32, (CH, 1), 0)
            acc = acc_ref[pl.ds(row0, CH), :]
            for g in range(sb * SB, sb * SB + SB):
                s = starts_ref[k * GB + g]
                e = starts_ref[k * GB + g + 1]
                y = lax.dot_general(
                    d, _round_bf16(t_ref[g]), (((1,), (1,)), ((), ())),
                    preferred_element_type=jnp.float32,
                )
                a = jnp.abs(y + rel_ref[g, :][None, :])
                m = (gl >= s) & (gl < e)
                acc = acc + jnp.where(m, a, 0.0)
            acc_ref[pl.ds(row0, CH), :] = acc
            return 0

        lax.fori_loop(lo // CH, (hi + CH - 1) // CH, chunk, 0)


def _tc_part0_body(starts_ref, t_ref, rel_ref, h_ref, tr_ref,
                   acc_ref, d_ref):
    B = h_ref.shape[0]
    k = pl.program_id(0)

    @pl.when(k == 0)
    def _():
        for c in range(B // CH):
            sl = pl.ds(c * CH, CH)
            d_ref[sl, :] = _round_bf16(h_ref[sl, :] - tr_ref[sl, :])
            acc_ref[sl, :] = jnp.zeros((CH, D_REL), jnp.float32)

    _score_blocks(starts_ref, t_ref, rel_ref, d_ref, acc_ref, k)


def _tc_part1_body(starts_ref, t_ref, rel_ref, acc_in_ref, d_in_ref,
                   out_ref, acc_ref):
    B = acc_in_ref.shape[0]
    k = pl.program_id(0)
    nsteps = pl.num_programs(0)

    @pl.when(k == 0)
    def _():
        for c in range(B // CH):
            sl = pl.ds(c * CH, CH)
            acc_ref[sl, :] = acc_in_ref[sl, :]

    _score_blocks(starts_ref, t_ref, rel_ref, d_in_ref, acc_ref, k)

    @pl.when(k == nsteps - 1)
    def _():
        for c in range(B // CH):
            sl = pl.ds(c * CH, CH)
            out_ref[sl, :] = jnp.sum(acc_ref[sl, :], axis=1, keepdims=True)


def _tc_score(t3a, t3b, rel, hrows, trows, starts):
    B = hrows.shape[0]
    na = t3a.shape[0]
    nb = t3b.shape[0]
    assert na % GB == 0 and nb % GB == 0

    spec0 = pltpu.PrefetchScalarGridSpec(
        num_scalar_prefetch=1,
        grid=(na // GB,),
        in_specs=[
            pl.BlockSpec((GB, D_REL, D_ENT), lambda k, st: (k, 0, 0)),
            pl.BlockSpec((GB, D_REL), lambda k, st: (k, 0)),
            pl.BlockSpec((B, D_ENT), lambda k, st: (0, 0)),
            pl.BlockSpec((B, D_ENT), lambda k, st: (0, 0)),
        ],
        out_specs=(
            pl.BlockSpec((B, D_REL), lambda k, st: (0, 0)),
            pl.BlockSpec((B, D_ENT), lambda k, st: (0, 0)),
        ),
    )
    acc, d = pl.pallas_call(
        _tc_part0_body,
        grid_spec=spec0,
        out_shape=(
            jax.ShapeDtypeStruct((B, D_REL), jnp.float32),
            jax.ShapeDtypeStruct((B, D_ENT), jnp.bfloat16),
        ),
    )(starts[: na + 1], t3a, rel[:na], hrows, trows)

    spec1 = pltpu.PrefetchScalarGridSpec(
        num_scalar_prefetch=1,
        grid=(nb // GB,),
        in_specs=[
            pl.BlockSpec((GB, D_REL, D_ENT), lambda k, st: (k, 0, 0)),
            pl.BlockSpec((GB, D_REL), lambda k, st: (k, 0)),
            pl.BlockSpec((B, D_REL), lambda k, st: (0, 0)),
            pl.BlockSpec((B, D_ENT), lambda k, st: (0, 0)),
        ],
        out_specs=pl.BlockSpec((B, 1), lambda k, st: (0, 0)),
        scratch_shapes=[
            pltpu.VMEM((B, D_REL), jnp.float32),
        ],
    )
    return pl.pallas_call(
        _tc_part1_body,
        grid_spec=spec1,
        out_shape=jax.ShapeDtypeStruct((B, 1), jnp.float32),
    )(starts[na:], t3b, rel[na:], acc, d)


def kernel(predict_h, predict_t, predict_r, ent_embeddings, rel_embeddings,
           transfer_matrix):
    B = predict_h.shape[0]
    nrel = rel_embeddings.shape[0]
    iota = jnp.arange(B, dtype=jnp.int32)
    _, h_s, t_s, perm = lax.sort(
        (predict_r * B + iota, predict_h, predict_t, iota), num_keys=1
    )
    rr = jnp.arange(nrel + 1, dtype=jnp.int32)
    starts = jnp.sum(
        (predict_r[None, :] < rr[:, None]).astype(jnp.int32), axis=1
    )
    hrows, trows = _sc_gather_entities(ent_embeddings, h_s, t_s)
    t3a = transfer_matrix[:SPLIT].reshape(SPLIT, D_REL, D_ENT)
    t3b = transfer_matrix[SPLIT:].reshape(nrel - SPLIT, D_REL, D_ENT)
    score_sorted = _tc_score(t3a, t3b, rel_embeddings, hrows, trows, starts)
    return jnp.zeros((B,), jnp.float32).at[perm].set(score_sorted[:, 0])[:, None]


# speedup vs baseline: 1.3441x; 1.1855x over previous
import functools

import jax
import jax.numpy as jnp
from jax import lax
from jax.experimental import pallas as pl
from jax.experimental.pallas import tpu as pltpu
from jax.experimental.pallas import tpu_sc as plsc

D_ENT = 128
D_REL = 128
NC = 2
NS = 16
GB = 40
SB = 8
CH = 128


def _round_bf16(x):
    xi = lax.bitcast_convert_type(x, jnp.uint32)
    xr = lax.bitcast_convert_type(xi + jnp.uint32(0x8000), jnp.float32)
    return xr.astype(jnp.bfloat16)


def _sc_gather_entities(ent, h_idx, t_idx):
    B = h_idx.shape[0]
    nw = NC * NS
    bpw = B // nw
    assert B % (8 * nw) == 0
    mesh = plsc.VectorSubcoreMesh(core_axis_name="c", subcore_axis_name="s")

    @functools.partial(
        pl.kernel,
        out_type=(
            jax.ShapeDtypeStruct((B, D_ENT), jnp.float32),
            jax.ShapeDtypeStruct((B, D_ENT), jnp.float32),
        ),
        mesh=mesh,
        scratch_types=[
            pltpu.VMEM((bpw,), jnp.int32),
            pltpu.VMEM((bpw,), jnp.int32),
            pltpu.VMEM((bpw, D_ENT), jnp.float32),
            pltpu.VMEM((bpw, D_ENT), jnp.float32),
            pltpu.SemaphoreType.DMA,
            pltpu.SemaphoreType.DMA,
        ],
    )
    def k(ent_hbm, h_hbm, t_hbm, hout, tout, hi_v, ti_v, hr_v, tr_v, s1, s2):
        wid = lax.axis_index("s") * NC + lax.axis_index("c")
        base = wid * bpw
        pltpu.sync_copy(h_hbm.at[wid], hi_v)
        pltpu.sync_copy(t_hbm.at[wid], ti_v)
        c1 = pltpu.async_copy(ent_hbm.at[hi_v], hr_v, s1)
        c2 = pltpu.async_copy(ent_hbm.at[ti_v], tr_v, s2)
        c1.wait()
        c2.wait()
        pltpu.sync_copy(hr_v, hout.at[pl.ds(base, bpw)])
        pltpu.sync_copy(tr_v, tout.at[pl.ds(base, bpw)])

    return k(ent, h_idx.reshape(nw, bpw), t_idx.reshape(nw, bpw))


def _score_blocks(starts_ref, t_ref, rel_ref, d_ref, acc_ref, k):
    for sb in range(GB // SB):
        lo = starts_ref[k * GB + sb * SB]
        hi = starts_ref[k * GB + sb * SB + SB]

        def chunk(c, _, sb=sb):
            row0 = pl.multiple_of(c * CH, CH)
            d = d_ref[pl.ds(row0, CH), :]
            gl = row0 + lax.broadcasted_iota(jnp.int32, (CH, 1), 0)
            acc = acc_ref[pl.ds(row0, CH), :]
            for g in range(sb * SB, sb * SB + SB):
                s = starts_ref[k * GB + g]
                e = starts_ref[k * GB + g + 1]
                y = lax.dot_general(
                    d, _round_bf16(t_ref[g]), (((1,), (1,)), ((), ())),
                    preferred_element_type=jnp.float32,
                )
                a = jnp.abs(y + rel_ref[g, :][None, :])
                m = (gl >= s) & (gl < e)
                acc = acc + jnp.where(m, a, 0.0)
            acc_ref[pl.ds(row0, CH), :] = acc
            return 0

        lax.fori_loop(lo // CH, (hi + CH - 1) // CH, chunk, 0)


def _tc_score_body(starts_ref, t_ref, rel_ref, h_ref, tr_ref, out_ref,
                   d_ref, acc_ref):
    B = h_ref.shape[0]
    k = pl.program_id(0)
    nsteps = pl.num_programs(0)

    @pl.when(k == 0)
    def _():
        for c in range(B // CH):
            sl = pl.ds(c * CH, CH)
            d_ref[sl, :] = _round_bf16(h_ref[sl, :] - tr_ref[sl, :])
            acc_ref[sl, :] = jnp.zeros((CH, D_REL), jnp.float32)

    _score_blocks(starts_ref, t_ref, rel_ref, d_ref, acc_ref, k)

    @pl.when(k == nsteps - 1)
    def _():
        for c in range(B // CH):
            sl = pl.ds(c * CH, CH)
            out_ref[sl, :] = jnp.sum(acc_ref[sl, :], axis=1, keepdims=True)


def _tc_score(t3, rel, hrows, trows, starts):
    B = hrows.shape[0]
    nrel = rel.shape[0]
    assert nrel % GB == 0
    grid_spec = pltpu.PrefetchScalarGridSpec(
        num_scalar_prefetch=1,
        grid=(nrel // GB,),
        in_specs=[
            pl.BlockSpec((GB, D_REL, D_ENT), lambda k, st: (k, 0, 0)),
            pl.BlockSpec((GB, D_REL), lambda k, st: (k, 0)),
            pl.BlockSpec((B, D_ENT), lambda k, st: (0, 0)),
            pl.BlockSpec((B, D_ENT), lambda k, st: (0, 0)),
        ],
        out_specs=pl.BlockSpec((B, 1), lambda k, st: (0, 0)),
        scratch_shapes=[
            pltpu.VMEM((B, D_ENT), jnp.bfloat16),
            pltpu.VMEM((B, D_REL), jnp.float32),
        ],
    )
    return pl.pallas_call(
        _tc_score_body,
        grid_spec=grid_spec,
        out_shape=jax.ShapeDtypeStruct((B, 1), jnp.float32),
    )(starts, t3, rel, hrows, trows)


def kernel(predict_h, predict_t, predict_r, ent_embeddings, rel_embeddings,
           transfer_matrix):
    B = predict_h.shape[0]
    nrel = rel_embeddings.shape[0]
    iota = jnp.arange(B, dtype=jnp.int32)
    _, h_s, t_s, perm = lax.sort(
        (predict_r * B + iota, predict_h, predict_t, iota), num_keys=1
    )
    rr = jnp.arange(nrel + 1, dtype=jnp.int32)
    starts = jnp.sum(
        (predict_r[None, :] < rr[:, None]).astype(jnp.int32), axis=1
    )
    hrows, trows = _sc_gather_entities(ent_embeddings, h_s, t_s)
    t3 = transfer_matrix.reshape(nrel, D_REL, D_ENT)
    score_sorted = _tc_score(t3, rel_embeddings, hrows, trows, starts)
    return jnp.zeros((B,), jnp.float32).at[perm].set(score_sorted[:, 0])[:, None]


# speedup vs baseline: 1.4503x; 1.0790x over previous
import functools

import jax
import jax.numpy as jnp
from jax import lax
from jax.experimental import pallas as pl
from jax.experimental.pallas import tpu as pltpu
from jax.experimental.pallas import tpu_sc as plsc

D_ENT = 128
D_REL = 128
NC = 2
NS = 16
GB = 40
SB = 8
CH = 128


def _round_bf16(x):
    xi = lax.bitcast_convert_type(x, jnp.uint32)
    xr = lax.bitcast_convert_type(xi + jnp.uint32(0x8000), jnp.float32)
    return xr.astype(jnp.bfloat16)


def _sc_gather_entities(ent, h_idx, t_idx):
    B = h_idx.shape[0]
    nw = NC * NS
    bpw = B // nw
    assert B % (8 * nw) == 0
    mesh = plsc.VectorSubcoreMesh(core_axis_name="c", subcore_axis_name="s")

    @functools.partial(
        pl.kernel,
        out_type=(
            jax.ShapeDtypeStruct((B, D_ENT), jnp.float32),
            jax.ShapeDtypeStruct((B, D_ENT), jnp.float32),
        ),
        mesh=mesh,
        scratch_types=[
            pltpu.VMEM((bpw,), jnp.int32),
            pltpu.VMEM((bpw,), jnp.int32),
            pltpu.VMEM((bpw, D_ENT), jnp.float32),
            pltpu.VMEM((bpw, D_ENT), jnp.float32),
            pltpu.SemaphoreType.DMA,
            pltpu.SemaphoreType.DMA,
        ],
    )
    def k(ent_hbm, h_hbm, t_hbm, hout, tout, hi_v, ti_v, hr_v, tr_v, s1, s2):
        wid = lax.axis_index("s") * NC + lax.axis_index("c")
        base = wid * bpw
        pltpu.sync_copy(h_hbm.at[pl.ds(base, bpw)], hi_v)
        pltpu.sync_copy(t_hbm.at[pl.ds(base, bpw)], ti_v)
        c1 = pltpu.async_copy(ent_hbm.at[hi_v], hr_v, s1)
        c2 = pltpu.async_copy(ent_hbm.at[ti_v], tr_v, s2)
        c1.wait()
        c2.wait()
        pltpu.sync_copy(hr_v, hout.at[pl.ds(base, bpw)])
        pltpu.sync_copy(tr_v, tout.at[pl.ds(base, bpw)])

    return k(ent, h_idx, t_idx)


def _score_blocks(starts_ref, t_ref, rel_ref, d_ref, acc_ref, k):
    for sb in range(GB // SB):
        lo = starts_ref[k * GB + sb * SB]
        hi = starts_ref[k * GB + sb * SB + SB]

        def chunk(c, _, sb=sb):
            row0 = pl.multiple_of(c * CH, CH)
            d = d_ref[pl.ds(row0, CH), :]
            gl = row0 + lax.broadcasted_iota(jnp.int32, (CH, 1), 0)
            acc = acc_ref[pl.ds(row0, CH), :]
            for g in range(sb * SB, sb * SB + SB):
                s = starts_ref[k * GB + g]
                e = starts_ref[k * GB + g + 1]
                y = lax.dot_general(
                    d, _round_bf16(t_ref[g]), (((1,), (1,)), ((), ())),
                    preferred_element_type=jnp.float32,
                )
                a = jnp.abs(y + rel_ref[g, :][None, :])
                m = (gl >= s) & (gl < e)
                acc = acc + jnp.where(m, a, 0.0)
            acc_ref[pl.ds(row0, CH), :] = acc
            return 0

        lax.fori_loop(lo // CH, (hi + CH - 1) // CH, chunk, 0)


def _tc_score_body(starts_ref, t_ref, rel_ref, h_ref, tr_ref, out_ref,
                   d_ref, acc_ref):
    B = h_ref.shape[0]
    k = pl.program_id(0)
    nsteps = pl.num_programs(0)

    @pl.when(k == 0)
    def _():
        for c in range(B // CH):
            sl = pl.ds(c * CH, CH)
            d_ref[sl, :] = _round_bf16(h_ref[sl, :] - tr_ref[sl, :])
            acc_ref[sl, :] = jnp.zeros((CH, D_REL), jnp.float32)

    _score_blocks(starts_ref, t_ref, rel_ref, d_ref, acc_ref, k)

    @pl.when(k == nsteps - 1)
    def _():
        for c in range(B // CH):
            sl = pl.ds(c * CH, CH)
            out_ref[sl, :] = jnp.sum(acc_ref[sl, :], axis=1, keepdims=True)


def _tc_score(t3, rel, hrows, trows, starts):
    B = hrows.shape[0]
    nrel = rel.shape[0]
    assert nrel % GB == 0
    grid_spec = pltpu.PrefetchScalarGridSpec(
        num_scalar_prefetch=1,
        grid=(nrel // GB,),
        in_specs=[
            pl.BlockSpec((GB, D_REL, D_ENT), lambda k, st: (k, 0, 0)),
            pl.BlockSpec((GB, D_REL), lambda k, st: (k, 0)),
            pl.BlockSpec((B, D_ENT), lambda k, st: (0, 0)),
            pl.BlockSpec((B, D_ENT), lambda k, st: (0, 0)),
        ],
        out_specs=pl.BlockSpec((B, 1), lambda k, st: (0, 0)),
        scratch_shapes=[
            pltpu.VMEM((B, D_ENT), jnp.bfloat16),
            pltpu.VMEM((B, D_REL), jnp.float32),
        ],
    )
    return pl.pallas_call(
        _tc_score_body,
        grid_spec=grid_spec,
        out_shape=jax.ShapeDtypeStruct((B, 1), jnp.float32),
    )(starts, t3, rel, hrows, trows)


def kernel(predict_h, predict_t, predict_r, ent_embeddings, rel_embeddings,
           transfer_matrix):
    B = predict_h.shape[0]
    nrel = rel_embeddings.shape[0]
    iota = jnp.arange(B, dtype=jnp.int32)
    _, h_s, t_s, perm = lax.sort(
        (predict_r * B + iota, predict_h, predict_t, iota), num_keys=1
    )
    rr = jnp.arange(nrel + 1, dtype=jnp.int32)
    starts = jnp.sum(
        (predict_r[None, :] < rr[:, None]).astype(jnp.int32), axis=1
    )
    hrows, trows = _sc_gather_entities(ent_embeddings, h_s, t_s)
    _, inv_perm = lax.sort((perm, iota), num_keys=1)
    t3 = transfer_matrix.reshape(nrel, D_REL, D_ENT)
    score_sorted = _tc_score(t3, rel_embeddings, hrows, trows, starts)
    return jnp.take(score_sorted, inv_perm, axis=0)
